# baseline (speedup 1.0000x reference)
import functools

import jax
import jax.numpy as jnp
from jax import lax
from jax.experimental import pallas as pl
from jax.experimental.pallas import tpu as pltpu
from jax.experimental.pallas import tpu_sc as plsc

B = 16384
L = 100
NC = 2
NS = 16
NW = NC * NS
RW = B // NW
C = 128
NCHUNK = RW // C
G = C // 16
CL = C * L
CH = CL // 2
VPAD = 1000192
VWORDS = VPAD // 2
TSLICE = 31232
HOP = 12800

_mesh = plsc.VectorSubcoreMesh(core_axis_name="c", subcore_axis_name="s")


@functools.partial(
    pl.kernel,
    mesh=_mesh,
    out_type=jax.ShapeDtypeStruct((B,), jnp.float32),
    scratch_types=[
        pltpu.VMEM_SHARED((VWORDS,), jnp.int32),
        pltpu.VMEM((L, C), jnp.int32),
        pltpu.VMEM((L, C), jnp.int32),
        pltpu.VMEM((CL,), jnp.int32),
        pltpu.VMEM((CL,), jnp.int32),
        pltpu.VMEM((CL,), jnp.int32),
        pltpu.VMEM((L, C), jnp.float32),
        pltpu.VMEM((RW,), jnp.float32),
        pltpu.VMEM((16,), jnp.int32),
        pltpu.VMEM((16,), jnp.float32),
        pltpu.SemaphoreType.DMA,
        pltpu.SemaphoreType.DMA,
        pltpu.SemaphoreType.DMA,
        pltpu.SemaphoreType.DMA,
        pltpu.SemaphoreType.DMA,
        pltpu.SemaphoreType.DMA,
    ],
    compiler_params=pltpu.CompilerParams(needs_layout_passes=False),
)
def _wide_sc(idx_hbm, val_hbm, tab_hbm, bias_hbm, out_hbm,
             tab_s, i2_0, i2_1, i1, g0, g1, v_v, o_v, zidx, bvec,
             si0, si1, sga, sgb, sv, sb):
    cid = lax.axis_index("c")
    sid = lax.axis_index("s")
    wid = sid * NC + cid
    lane = lax.iota(jnp.int32, 16)
    i2_b = (i2_0, i2_1)
    g_b = (g0, g1)
    si = (si0, si1)

    def start_i2(k):
        s = k % 2
        return pltpu.async_copy(
            idx_hbm.at[:, pl.ds(wid * RW + k * C, C)], i2_b[s], si[s])

    def start_val(k):
        return pltpu.async_copy(
            val_hbm.at[:, pl.ds(wid * RW + k * C, C)], v_v, sv)

    def start_gather(k):
        s = k % 2
        ca = pltpu.async_copy(
            tab_s.at[i1.at[pl.ds(0, CH)]], g_b[s].at[pl.ds(0, CH)], sga)
        cb = pltpu.async_copy(
            tab_s.at[i1.at[pl.ds(CH, CH)]], g_b[s].at[pl.ds(CH, CH)], sgb)
        return (ca, cb)

    def flatten(k):
        s = k % 2
        i2 = i2_b[s]

        def slot_body(l, carry):
            for cb in range(C // 16):
                x = i2[l, pl.ds(cb * 16, 16)]
                i1[pl.ds(l * C + cb * 16, 16)] = x >> 1
            return carry

        lax.fori_loop(0, L, slot_body, 0)

    cp_i2_0 = start_i2(0)
    cp_v = start_val(0)
    cp_i2 = start_i2(1)

    zidx[...] = lane * 0
    pltpu.async_copy(bias_hbm.at[zidx], bvec, sb).wait()

    tbase = sid * TSLICE
    for h in range(TSLICE // HOP):
        off = tbase + h * HOP
        pltpu.sync_copy(tab_hbm.at[pl.ds(off, HOP)], g0)
        pltpu.sync_copy(g0, tab_s.at[pl.ds(off, HOP)])
    rem = TSLICE - (TSLICE // HOP) * HOP
    if rem:
        off = tbase + (TSLICE // HOP) * HOP
        pltpu.sync_copy(tab_hbm.at[pl.ds(off, rem)], g0.at[pl.ds(0, rem)])
        pltpu.sync_copy(g0.at[pl.ds(0, rem)], tab_s.at[pl.ds(off, rem)])

    tail = VWORDS - NS * TSLICE

    @pl.when(sid == NS - 1)
    def _stage_tail():
        toff = NS * TSLICE
        pltpu.sync_copy(tab_hbm.at[pl.ds(toff, tail)], g0.at[pl.ds(0, tail)])
        pltpu.sync_copy(g0.at[pl.ds(0, tail)], tab_s.at[pl.ds(toff, tail)])

    plsc.subcore_barrier()

    cp_i2_0.wait()
    flatten(0)
    cp_g = start_gather(0)

    himask = jnp.full((16,), -65536, jnp.int32)

    for k in range(NCHUNK):
        s = k % 2
        cp_g[0].wait()
        cp_g[1].wait()
        if k + 1 < NCHUNK:
            cp_i2.wait()
            flatten(k + 1)
            cp_g = start_gather(k + 1)
        cp_v.wait()

        g_v, i2_v = g_b[s], i2_b[s]

        def grp_body(g, carry2, g_v=g_v, v_v=v_v, i2_v=i2_v, k=k):
            acc = bvec[...]
            for slot in range(L):
                w = g_v[pl.ds(slot * C + g * 16, 16)]
                par = i2_v[slot, pl.ds(g * 16, 16)] & 1
                bits = jnp.where(par == 1, w & himask, w << 16)
                gv = plsc.bitcast(bits, jnp.float32)
                vv = v_v[slot, pl.ds(g * 16, 16)]
                acc = acc + gv * vv
            o_v[pl.ds(k * C + g * 16, 16)] = acc
            return carry2

        lax.fori_loop(0, G, grp_body, 0)

        if k + 1 < NCHUNK:
            cp_v = start_val(k + 1)
        if k + 2 < NCHUNK:
            cp_i2 = start_i2(k + 2)

    pltpu.sync_copy(o_v, out_hbm.at[pl.ds(wid * RW, RW)])


def kernel(indices, values, emb_table, bias):
    idx_t = indices.astype(jnp.int32).T
    val_t = values.T
    u16 = jax.lax.bitcast_convert_type(
        emb_table.astype(jnp.bfloat16), jnp.uint16).reshape(-1)
    u16 = jnp.pad(u16, (0, VPAD - u16.shape[0]))
    w = u16[0::2].astype(jnp.uint32) | (u16[1::2].astype(jnp.uint32) << 16)
    tab = jax.lax.bitcast_convert_type(w, jnp.int32)
    return _wide_sc(idx_t, val_t, tab, bias)


# speedup vs baseline: 2.0360x; 2.0360x over previous
import functools

import jax
import jax.numpy as jnp
from jax import lax
from jax.experimental import pallas as pl
from jax.experimental.pallas import tpu as pltpu
from jax.experimental.pallas import tpu_sc as plsc

B = 16384
L = 100
NC = 2
NS = 16
NW = NC * NS
RW = B // NW
C = 128
NCHUNK = RW // C
G = C // 16
CL = C * L
CH = CL // 2

_mesh = plsc.VectorSubcoreMesh(core_axis_name="c", subcore_axis_name="s")


@functools.partial(
    pl.kernel,
    mesh=_mesh,
    out_type=jax.ShapeDtypeStruct((B,), jnp.float32),
    scratch_types=[
        pltpu.VMEM((L, C), jnp.int32),
        pltpu.VMEM((L, C), jnp.int32),
        pltpu.VMEM((CL,), jnp.int32),
        pltpu.VMEM((CL,), jnp.int32),
        pltpu.VMEM((CL,), jnp.float32),
        pltpu.VMEM((CL,), jnp.float32),
        pltpu.VMEM((L, C), jnp.float32),
        pltpu.VMEM((L, C), jnp.float32),
        pltpu.VMEM((RW,), jnp.float32),
        pltpu.VMEM((16,), jnp.int32),
        pltpu.VMEM((16,), jnp.float32),
        pltpu.SemaphoreType.DMA,
        pltpu.SemaphoreType.DMA,
        pltpu.SemaphoreType.DMA,
        pltpu.SemaphoreType.DMA,
        pltpu.SemaphoreType.DMA,
        pltpu.SemaphoreType.DMA,
        pltpu.SemaphoreType.DMA,
        pltpu.SemaphoreType.DMA,
        pltpu.SemaphoreType.DMA,
    ],
    compiler_params=pltpu.CompilerParams(needs_layout_passes=False),
)
def _wide_sc(idx_hbm, val_hbm, tab_hbm, bias_hbm, out_hbm,
             i2_0, i2_1, idx0, idx1, g0, g1, v0, v1, o_v, zidx, bvec,
             si0, si1, sg0, sg1, sga0, sga1, sv0, sv1, sb):
    cid = lax.axis_index("c")
    sid = lax.axis_index("s")
    wid = sid * NC + cid
    lane = lax.iota(jnp.int32, 16)
    i2_b = (i2_0, i2_1)
    idx_b = (idx0, idx1)
    g_b = (g0, g1)
    v_b = (v0, v1)
    si = (si0, si1)
    sg = (sg0, sg1)
    sga = (sga0, sga1)
    sv = (sv0, sv1)

    def start_i2(k):
        s = k % 2
        return pltpu.async_copy(
            idx_hbm.at[:, pl.ds(wid * RW + k * C, C)], i2_b[s], si[s])

    def start_val(k):
        s = k % 2
        return pltpu.async_copy(
            val_hbm.at[:, pl.ds(wid * RW + k * C, C)], v_b[s], sv[s])

    def start_gather(k):
        s = k % 2
        ca = pltpu.async_copy(
            tab_hbm.at[idx_b[s].at[pl.ds(0, CH)]], g_b[s].at[pl.ds(0, CH)],
            sg[s])
        cb = pltpu.async_copy(
            tab_hbm.at[idx_b[s].at[pl.ds(CH, CH)]], g_b[s].at[pl.ds(CH, CH)],
            sga[s])
        return (ca, cb)

    def flatten(k):
        s = k % 2
        i2, i1 = i2_b[s], idx_b[s]

        def slot_body(l, carry):
            for cb in range(C // 16):
                i1[pl.ds(l * C + cb * 16, 16)] = i2[l, pl.ds(cb * 16, 16)]
            return carry

        lax.fori_loop(0, L, slot_body, 0)

    cp_i2_0 = start_i2(0)
    cp_v = start_val(0)
    cp_i2 = start_i2(1)

    zidx[...] = lane * 0
    pltpu.async_copy(bias_hbm.at[zidx], bvec, sb).wait()

    cp_i2_0.wait()
    flatten(0)
    cp_g = start_gather(0)

    for k in range(NCHUNK):
        s = k % 2
        if k + 1 < NCHUNK:
            cp_i2.wait()
            flatten(k + 1)
        cp_g[0].wait()
        cp_g[1].wait()
        if k + 1 < NCHUNK:
            cp_g = start_gather(k + 1)
        if k + 2 < NCHUNK:
            cp_i2 = start_i2(k + 2)
        cp_v.wait()
        if k + 1 < NCHUNK:
            cp_v = start_val(k + 1)

        g_v, v_v = g_b[s], v_b[s]

        def grp_body(g, carry2, g_v=g_v, v_v=v_v, k=k):
            acc = bvec[...]
            for slot in range(L):
                gv = g_v[pl.ds(slot * C + g * 16, 16)]
                vv = v_v[slot, pl.ds(g * 16, 16)]
                acc = acc + gv * vv
            o_v[pl.ds(k * C + g * 16, 16)] = acc
            return carry2

        lax.fori_loop(0, G, grp_body, 0)

    pltpu.sync_copy(o_v, out_hbm.at[pl.ds(wid * RW, RW)])


def kernel(indices, values, emb_table, bias):
    idx_t = indices.astype(jnp.int32).T
    val_t = values.T
    tab = emb_table.reshape(-1)
    return _wide_sc(idx_t, val_t, tab, bias)
